# Initial kernel scaffold; baseline (speedup 1.0000x reference)
import jax
import jax.numpy as jnp
from jax.experimental import pallas as pl


def kernel(x, edge_index, edge_weight, W, b):
    raise NotImplementedError("write your pallas kernel here")


# speedup vs baseline: 4.5646x; 4.5646x over previous
import functools

import jax
import jax.numpy as jnp
from jax import lax
from jax.experimental import pallas as pl
from jax.experimental.pallas import tpu as pltpu
from jax.experimental.pallas import tpu_sc as plsc

N_NODES = 10000
N_EDGES = 160000
D = 256
DH = 128
NSUB = 16
NCORE = 2
CHUNK = 128
EPT = 10240
NCHUNK = EPT // CHUNK
PAD_EDGES = NSUB * EPT
ROWS_PAD = 10240
RPT = ROWS_PAD // NSUB


def _sc_aggregate(x2, src3, dst3, w3):
  mesh = plsc.VectorSubcoreMesh(core_axis_name="c", subcore_axis_name="s")

  @functools.partial(
      pl.kernel,
      out_type=jax.ShapeDtypeStruct((NCORE, ROWS_PAD, DH), jnp.float32),
      mesh=mesh,
      scratch_types=[
          pltpu.VMEM((NCHUNK, CHUNK), jnp.int32),
          pltpu.VMEM((NCHUNK, CHUNK), jnp.int32),
          pltpu.VMEM((NCHUNK, CHUNK), jnp.float32),
          pltpu.VMEM((CHUNK,), jnp.int32),
          pltpu.VMEM((CHUNK,), jnp.int32),
          pltpu.VMEM((CHUNK, DH), jnp.float32),
          pltpu.VMEM_SHARED((ROWS_PAD, DH), jnp.float32),
          pltpu.SemaphoreType.DMA,
      ],
  )
  def k(x_hbm, src_hbm, dst_hbm, w_hbm, out_hbm,
        src_v, dst_v, w_v, sidx_v, didx_v, rows_v, acc_sh, sem):
    c = lax.axis_index("c")
    s = lax.axis_index("s")
    wid = c * NSUB + s

    pltpu.sync_copy(src_hbm.at[wid], src_v)
    pltpu.sync_copy(dst_hbm.at[s], dst_v)
    pltpu.sync_copy(w_hbm.at[s], w_v)

    def zrow(i, carry):
      for v in range(DH // 16):
        rows_v[i, pl.ds(v * 16, 16)] = jnp.zeros((16,), jnp.float32)
      return carry
    lax.fori_loop(0, CHUNK, zrow, 0)
    for r in range(RPT // CHUNK):
      pltpu.sync_copy(rows_v, acc_sh.at[pl.ds(s * RPT + r * CHUNK, CHUNK)])
    plsc.subcore_barrier()

    def chunk(j, carry):
      for v in range(CHUNK // 16):
        sl = pl.ds(v * 16, 16)
        sidx_v[sl] = src_v[j, sl]
        didx_v[sl] = dst_v[j, sl]
      pltpu.async_copy(x_hbm.at[sidx_v], rows_v, sem).wait()
      def scale(g, cc):
        w16 = w_v[j, pl.ds(g * 16, 16)]
        for l in range(16):
          wl = w16[l]
          e = g * 16 + l
          for v in range(DH // 16):
            sl = pl.ds(v * 16, 16)
            rows_v[e, sl] = rows_v[e, sl] * wl
        return cc
      lax.fori_loop(0, CHUNK // 16, scale, 0)
      pltpu.sync_copy(rows_v, acc_sh.at[didx_v], add=True)
      return carry
    lax.fori_loop(0, NCHUNK, chunk, 0)

    plsc.subcore_barrier()
    pltpu.sync_copy(acc_sh.at[pl.ds(s * RPT, RPT)],
                    out_hbm.at[c, pl.ds(s * RPT, RPT)])

  return k(x2, src3, dst3, w3)


def _mm_body(a_ref, w_ref, b_ref, o_ref):
  acc = jnp.dot(a_ref[0], w_ref[0], preferred_element_type=jnp.float32)
  acc = acc + jnp.dot(a_ref[1], w_ref[1], preferred_element_type=jnp.float32)
  o_ref[...] = jax.nn.sigmoid(acc + b_ref[...])


def _tc_epilogue(agg, W2, b2):
  blk = 512
  grid = ROWS_PAD // blk
  return pl.pallas_call(
      _mm_body,
      grid=(grid,),
      in_specs=[
          pl.BlockSpec((2, blk, DH), lambda i: (0, i, 0)),
          pl.BlockSpec((2, DH, D), lambda i: (0, 0, 0)),
          pl.BlockSpec((1, D), lambda i: (0, 0)),
      ],
      out_specs=pl.BlockSpec((blk, D), lambda i: (i, 0)),
      out_shape=jax.ShapeDtypeStruct((ROWS_PAD, D), jnp.float32),
  )(agg, W2, b2)


def kernel(x, edge_index, edge_weight, W, b):
  src = edge_index[0]
  dst = edge_index[1]
  pad = PAD_EDGES - N_EDGES
  pad_ar = jnp.arange(pad, dtype=jnp.int32)
  src_p = jnp.concatenate([src, pad_ar % N_NODES])
  dst_p = jnp.concatenate([dst, N_NODES + pad_ar % (ROWS_PAD - N_NODES)])
  w_p = jnp.concatenate([edge_weight, jnp.zeros((pad,), jnp.float32)])

  src_r = src_p.reshape(NSUB, NCHUNK, CHUNK)
  src3 = jnp.concatenate([src_r, src_r + N_NODES], axis=0)
  dst3 = dst_p.reshape(NSUB, NCHUNK, CHUNK)
  w3 = w_p.reshape(NSUB, NCHUNK, CHUNK)
  x2 = x.reshape(N_NODES, 2, DH).transpose(1, 0, 2).reshape(2 * N_NODES, DH)

  agg = _sc_aggregate(x2, src3, dst3, w3)
  out = _tc_epilogue(agg, W.reshape(2, DH, D), b.reshape(1, D))
  return out[:N_NODES]


# speedup vs baseline: 7.4746x; 1.6375x over previous
import functools

import jax
import jax.numpy as jnp
from jax import lax
from jax.experimental import pallas as pl
from jax.experimental.pallas import tpu as pltpu
from jax.experimental.pallas import tpu_sc as plsc

N_NODES = 10000
N_EDGES = 160000
D = 256
DH = 128
NSUB = 16
NCORE = 2
CHUNK = 80
NBUF = 4
NCHUNK = 128
EPT = NCHUNK * CHUNK
PAD_EDGES = NSUB * EPT
RPT = N_NODES // NSUB
NSTRIP = DH // 16
NGRP = CHUNK // 16


def _sc_aggregate(x2, edata, wdata):
  mesh = plsc.VectorSubcoreMesh(core_axis_name="c", subcore_axis_name="s")

  @functools.partial(
      pl.kernel,
      out_type=jax.ShapeDtypeStruct((NCORE, N_NODES, DH), jnp.float32),
      mesh=mesh,
      scratch_types=[
          [pltpu.VMEM((2, CHUNK), jnp.int32)] * NBUF,
          [pltpu.VMEM((CHUNK,), jnp.float32)] * NBUF,
          [pltpu.VMEM((CHUNK,), jnp.int32)] * NBUF,
          [pltpu.VMEM((CHUNK, DH), jnp.float32)] * NBUF,
          pltpu.VMEM_SHARED((N_NODES, DH), jnp.float32),
          [pltpu.SemaphoreType.DMA] * NBUF,
          [pltpu.SemaphoreType.DMA] * NBUF,
          [pltpu.SemaphoreType.DMA] * NBUF,
          [pltpu.SemaphoreType.DMA] * NBUF,
      ],
  )
  def k(x_hbm, e_hbm, w_hbm, out_hbm, ech, wch, didx, rows, acc_sh,
        esem, wsem, gsem, ssem):
    c = lax.axis_index("c")
    s = lax.axis_index("s")
    wid = c * NSUB + s

    def zrow(i, carry):
      for v in range(NSTRIP):
        rows[0][i, pl.ds(v * 16, 16)] = jnp.zeros((16,), jnp.float32)
      return carry
    lax.fori_loop(0, CHUNK, zrow, 0)
    base = s * RPT
    for r in range(RPT // CHUNK):
      pltpu.sync_copy(rows[0], acc_sh.at[pl.ds(base + r * CHUNK, CHUNK)])
    rem = RPT - (RPT // CHUNK) * CHUNK
    pltpu.sync_copy(rows[0].at[pl.ds(0, rem)],
                    acc_sh.at[pl.ds(base + RPT - rem, rem)])
    plsc.subcore_barrier()

    def fetch_edata(b, j):
      pltpu.async_copy(e_hbm.at[wid, j], ech[b], esem[b])
      pltpu.async_copy(w_hbm.at[s, j], wch[b], wsem[b])

    def wait_edata(b, j):
      pltpu.make_async_copy(e_hbm.at[wid, j], ech[b], esem[b]).wait()
      pltpu.make_async_copy(w_hbm.at[s, j], wch[b], wsem[b]).wait()

    def issue_gather(b):
      pltpu.async_copy(x_hbm.at[ech[b].at[0]], rows[b], gsem[b])

    def wait_gather(b):
      pltpu.make_async_copy(x_hbm.at[ech[b].at[0]], rows[b], gsem[b]).wait()

    def issue_scatter(b):
      pltpu.async_copy(rows[b], acc_sh.at[didx[b]], ssem[b], add=True)

    def wait_scatter(b):
      pltpu.make_async_copy(rows[b], acc_sh.at[didx[b]], ssem[b]).wait()

    def scale(b, j):
      def grp(g, cc):
        w16 = wch[b][pl.ds(g * 16, 16)]
        for l in range(16):
          wl = w16[l]
          e = g * 16 + l
          for v in range(NSTRIP):
            sl = pl.ds(v * 16, 16)
            rows[b][e, sl] = rows[b][e, sl] * wl
        return cc
      lax.fori_loop(0, NGRP, grp, 0)

    for j in range(3):
      fetch_edata(j, j)
    for j in range(2):
      wait_edata(j, j)
      issue_gather(j)

    def quad(q, cc):
      for b in range(NBUF):
        j = q * NBUF + b
        b2 = (b + 2) % NBUF
        b3 = (b + 3) % NBUF

        @pl.when(j + 3 < NCHUNK)
        def _fetch():
          fetch_edata(b3, j + 3)

        wait_gather(b)
        for v in range(NGRP):
          sl = pl.ds(v * 16, 16)
          didx[b][sl] = ech[b][1, sl]
        scale(b, j)
        issue_scatter(b)

        @pl.when(j >= 2)
        def _drain():
          wait_scatter(b2)

        @pl.when(j + 2 < NCHUNK)
        def _gather():
          wait_edata(b2, j + 2)
          issue_gather(b2)
      return cc
    lax.fori_loop(0, NCHUNK // NBUF, quad, 0)
    wait_scatter(NBUF - 2)
    wait_scatter(NBUF - 1)

    plsc.subcore_barrier()
    @pl.when(s < NSUB - 1)
    def _full():
      pltpu.sync_copy(acc_sh.at[pl.ds(s * 640, 640)],
                      out_hbm.at[c, pl.ds(s * 640, 640)])

    @pl.when(s == NSUB - 1)
    def _last():
      pltpu.sync_copy(acc_sh.at[pl.ds((NSUB - 1) * 640, 400)],
                      out_hbm.at[c, pl.ds((NSUB - 1) * 640, 400)])

  return k(x2, edata, wdata)


def _mm_body(a_ref, w_ref, b_ref, o_ref):
  acc = jnp.dot(a_ref[0], w_ref[0], preferred_element_type=jnp.float32)
  acc = acc + jnp.dot(a_ref[1], w_ref[1], preferred_element_type=jnp.float32)
  o_ref[...] = jax.nn.sigmoid(acc + b_ref[...])


def _tc_epilogue(agg, W2, b2):
  blk = 1000
  grid = N_NODES // blk
  return pl.pallas_call(
      _mm_body,
      grid=(grid,),
      in_specs=[
          pl.BlockSpec((2, blk, DH), lambda i: (0, i, 0)),
          pl.BlockSpec((2, DH, D), lambda i: (0, 0, 0)),
          pl.BlockSpec((1, D), lambda i: (0, 0)),
      ],
      out_specs=pl.BlockSpec((blk, D), lambda i: (i, 0)),
      out_shape=jax.ShapeDtypeStruct((N_NODES, D), jnp.float32),
  )(agg, W2, b2)


def kernel(x, edge_index, edge_weight, W, b):
  src = edge_index[0]
  dst = edge_index[1]
  pad = PAD_EDGES - N_EDGES
  pad_ar = jnp.arange(pad, dtype=jnp.int32)
  src_p = jnp.concatenate([src, pad_ar % N_NODES])
  dst_p = jnp.concatenate([dst, pad_ar % N_NODES])
  w_p = jnp.concatenate([edge_weight, jnp.zeros((pad,), jnp.float32)])

  src_r = src_p.reshape(NSUB, NCHUNK, CHUNK)
  dst_r = dst_p.reshape(NSUB, NCHUNK, CHUNK)
  wdata = w_p.reshape(NSUB, NCHUNK, CHUNK)
  e0 = jnp.stack([src_r, dst_r], axis=2)
  e1 = jnp.stack([src_r + N_NODES, dst_r], axis=2)
  edata = jnp.concatenate([e0, e1], axis=0)
  x2 = x.reshape(N_NODES, 2, DH).transpose(1, 0, 2).reshape(2 * N_NODES, DH)

  agg = _sc_aggregate(x2, edata, wdata)
  out = _tc_epilogue(agg, W.reshape(2, DH, D), b.reshape(1, D))
  return out
